# baseline (speedup 1.0000x reference)
import jax
import jax.numpy as jnp
from jax.experimental import pallas as pl
from jax.experimental.pallas import tpu as pltpu

_BT = 8
_NEG = float(jnp.finfo(jnp.float32).min)


def _gelu(x):
    u = x * (0.7978845608028654 + 0.03567740814183417 * (x * x))
    h = 0.5 * x
    return h + h * jnp.tanh(u)


def _fused_kernel(xa_ref, xb_ref,
                  w1_ref, b1_ref, w2_ref, b2_ref, w3_ref, b3_ref,
                  w4_ref, b4_ref, w5_ref, b5_ref, w6_ref, b6_ref,
                  o_ref):
    BT = xa_ref.shape[0]
    f32 = jnp.float32

    heo = jnp.dot(xa_ref[...].reshape(BT * 256, 60), w1_ref[...],
                  preferred_element_type=f32) + b1_ref[...]
    heo = _gelu(heo).reshape(BT, 256, 128)

    m = jnp.maximum(heo[:, :250, :64], heo[:, :250, 64:])
    mp = jnp.pad(m, ((0, 0), (2, 2), (0, 0)), constant_values=_NEG)
    p1 = mp[:, 0:251, :]
    for a in range(1, 4):
        p1 = jnp.maximum(p1, mp[:, a:a + 251, :])

    p1z = jnp.pad(p1, ((0, 0), (4, 4), (0, 0)))
    win2 = jnp.concatenate([p1z[:, k:k + 252, :] for k in range(8)], axis=2)
    win2 = jnp.pad(win2, ((0, 0), (0, 4), (0, 0)))
    h2 = _gelu(jnp.dot(win2.reshape(BT * 256, 512), w2_ref[...],
                       preferred_element_type=f32) + b2_ref[...])
    h2 = h2.reshape(BT, 256, 128)

    h2z = jnp.pad(h2[:, :252, :], ((0, 0), (4, 4), (0, 0)))
    win3 = jnp.concatenate([h2z[:, k:k + 253, :] for k in range(8)], axis=2)
    win3 = jnp.pad(win3, ((0, 0), (0, 3), (0, 0)))
    h3 = _gelu(jnp.dot(win3.reshape(BT * 256, 1024), w3_ref[...],
                       preferred_element_type=f32) + b3_ref[...])
    h3 = h3.reshape(BT, 256, 128)

    h3p = jnp.pad(h3[:, :253, :], ((0, 0), (2, 1), (0, 0)),
                  constant_values=_NEG)
    x1 = jnp.max(h3p.reshape(BT, 64, 4, 128), axis=2)

    xb = xb_ref[...]
    win4 = jnp.concatenate([xb[:, a:a + 31, :] for a in range(5)], axis=2)
    win4 = jnp.pad(win4, ((0, 0), (0, 1), (0, 0)))
    geo = _gelu(jnp.dot(win4.reshape(BT * 32, 500), w4_ref[...],
                        preferred_element_type=f32) + b4_ref[...])
    geo = geo.reshape(BT, 32, 128)

    ge = geo[:, :31, :64]
    go = geo[:, :31, 64:]
    ridx = jax.lax.broadcasted_iota(jnp.int32, (BT, 31, 64), 1)
    m2 = jnp.maximum(ge, jnp.where(ridx < 30, go, _NEG))
    m2p = jnp.pad(m2, ((0, 0), (1, 0), (0, 0)), constant_values=_NEG)
    p4 = jnp.maximum(m2p[:, 0:31, :], m2p[:, 1:32, :])

    p4z = jnp.pad(p4, ((0, 0), (3, 3), (0, 0)))
    win5 = jnp.concatenate([p4z[:, k:k + 31, :] for k in range(7)], axis=2)
    win5 = jnp.pad(win5, ((0, 0), (0, 1), (0, 0)))
    g2 = _gelu(jnp.dot(win5.reshape(BT * 32, 448), w5_ref[...],
                       preferred_element_type=f32) + b5_ref[...])
    g2 = g2.reshape(BT, 32, 128)

    g2z = jnp.pad(g2[:, :31, :], ((0, 0), (3, 3), (0, 0)))
    win6 = jnp.concatenate([g2z[:, k:k + 31, :] for k in range(7)], axis=2)
    win6 = jnp.pad(win6, ((0, 0), (0, 1), (0, 0)))
    g3 = _gelu(jnp.dot(win6.reshape(BT * 32, 896), w6_ref[...],
                       preferred_element_type=f32) + b6_ref[...])
    g3 = g3.reshape(BT, 32, 128)

    g3p = jnp.pad(g3[:, :31, :], ((0, 0), (1, 0), (0, 0)),
                  constant_values=_NEG)
    x2 = jnp.max(g3p.reshape(BT, 16, 2, 128), axis=2)

    out_cl = jnp.concatenate([x1, x2], axis=1)
    o_ref[...] = jnp.transpose(out_cl, (0, 2, 1))


def _fold_bn(gamma, beta, mean, var, eps=1e-5):
    inv = gamma / jnp.sqrt(var + eps)
    return inv, beta - mean * inv


def kernel(x,
           b1c1_w, b1bn1_gamma, b1bn1_beta, b1bn1_mean, b1bn1_var,
           b1c2_w, b1bn2_gamma, b1bn2_beta, b1bn2_mean, b1bn2_var,
           b1c3_w, b1bn3_gamma, b1bn3_beta, b1bn3_mean, b1bn3_var,
           b2c1_w, b2bn1_gamma, b2bn1_beta, b2bn1_mean, b2bn1_var,
           b2c2_w, b2bn2_gamma, b2bn2_beta, b2bn2_mean, b2bn2_var,
           b2c3_w, b2bn3_gamma, b2bn3_beta, b2bn3_mean, b2bn3_var):
    B = x.shape[0]
    x2d = x.reshape(B, 3000)

    xa12 = jnp.pad(x2d, ((0, 0), (24, 24))).reshape(B, 254, 12)
    wa = jnp.concatenate([xa12[:, a:a + 250, :] for a in range(5)], axis=2)
    wa = jnp.pad(wa, ((0, 0), (0, 6), (0, 0)))
    xb = jnp.pad(x2d, ((0, 0), (200, 300))).reshape(B, 35, 100)

    s1, t1 = _fold_bn(b1bn1_gamma, b1bn1_beta, b1bn1_mean, b1bn1_var)
    w1 = b1c1_w[:, 0, :] * s1[:, None]
    w1eo = jnp.concatenate([jnp.pad(w1, ((0, 0), (0, 10))).T,
                            jnp.pad(w1, ((0, 0), (6, 4))).T], axis=1)
    b1eo = jnp.concatenate([t1, t1]).reshape(1, 128)

    s2, t2 = _fold_bn(b1bn2_gamma, b1bn2_beta, b1bn2_mean, b1bn2_var)
    w2f = b1c2_w.transpose(2, 1, 0).reshape(512, 128) * s2[None, :]
    b2v = t2.reshape(1, 128)

    s3, t3 = _fold_bn(b1bn3_gamma, b1bn3_beta, b1bn3_mean, b1bn3_var)
    w3f = b1c3_w.transpose(2, 1, 0).reshape(1024, 128) * s3[None, :]
    b3v = t3.reshape(1, 128)

    s4, t4 = _fold_bn(b2bn1_gamma, b2bn1_beta, b2bn1_mean, b2bn1_var)
    w4 = b2c1_w[:, 0, :] * s4[:, None]
    w4eo = jnp.concatenate([jnp.pad(w4, ((0, 0), (0, 100))).T,
                            jnp.pad(w4, ((0, 0), (50, 50))).T], axis=1)
    b4eo = jnp.concatenate([t4, t4]).reshape(1, 128)

    s5, t5 = _fold_bn(b2bn2_gamma, b2bn2_beta, b2bn2_mean, b2bn2_var)
    w5f = b2c2_w.transpose(2, 1, 0).reshape(448, 128) * s5[None, :]
    b5v = t5.reshape(1, 128)

    s6, t6 = _fold_bn(b2bn3_gamma, b2bn3_beta, b2bn3_mean, b2bn3_var)
    w6f = b2c3_w.transpose(2, 1, 0).reshape(896, 128) * s6[None, :]
    b6v = t6.reshape(1, 128)

    bcast = lambda i: (0, 0)
    return pl.pallas_call(
        _fused_kernel,
        out_shape=jax.ShapeDtypeStruct((B, 128, 80), x.dtype),
        grid=(B // _BT,),
        in_specs=[
            pl.BlockSpec((_BT, 256, 60), lambda i: (i, 0, 0)),
            pl.BlockSpec((_BT, 35, 100), lambda i: (i, 0, 0)),
            pl.BlockSpec((60, 128), bcast), pl.BlockSpec((1, 128), bcast),
            pl.BlockSpec((512, 128), bcast), pl.BlockSpec((1, 128), bcast),
            pl.BlockSpec((1024, 128), bcast), pl.BlockSpec((1, 128), bcast),
            pl.BlockSpec((500, 128), bcast), pl.BlockSpec((1, 128), bcast),
            pl.BlockSpec((448, 128), bcast), pl.BlockSpec((1, 128), bcast),
            pl.BlockSpec((896, 128), bcast), pl.BlockSpec((1, 128), bcast),
        ],
        out_specs=pl.BlockSpec((_BT, 128, 80), lambda i: (i, 0, 0)),
        compiler_params=pltpu.CompilerParams(
            dimension_semantics=("parallel",)),
    )(wa, xb, w1eo, b1eo, w2f, b2v, w3f, b3v, w4eo, b4eo, w5f, b5v, w6f, b6v)


# speedup vs baseline: 1.9007x; 1.9007x over previous
import jax
import jax.numpy as jnp
from jax.experimental import pallas as pl
from jax.experimental.pallas import tpu as pltpu

_BT = 8
_NEG = float(jnp.finfo(jnp.float32).min)


def _gelu(x):
    u = x * (0.7978845608028654 + 0.03567740814183417 * (x * x))
    h = 0.5 * x
    return h + h * jnp.tanh(u)


def _fused_kernel(xa_ref, xb_ref,
                  w1_ref, b1_ref, w2_ref, b2_ref, w3_ref, b3_ref,
                  w4_ref, b4_ref, w5_ref, b5_ref, w6_ref, b6_ref,
                  o_ref):
    BT = xa_ref.shape[0]
    f32 = jnp.float32

    xa = xa_ref[...]
    win1 = jnp.concatenate([xa[:, a:a + 250, :] for a in range(5)], axis=2)
    win1 = jnp.pad(win1, ((0, 0), (0, 6), (0, 0)))
    heo = jnp.dot(win1.reshape(BT * 256, 60), w1_ref[...],
                  preferred_element_type=f32) + b1_ref[...]
    heo = _gelu(heo).reshape(BT, 256, 128)

    m = jnp.maximum(heo[:, :250, :64], heo[:, :250, 64:])
    mp = jnp.pad(m, ((0, 0), (2, 2), (0, 0)), constant_values=_NEG)
    p1 = mp[:, 0:251, :]
    for a in range(1, 4):
        p1 = jnp.maximum(p1, mp[:, a:a + 251, :])

    p1z = jnp.pad(p1, ((0, 0), (4, 4), (0, 0)))
    win2 = jnp.concatenate([p1z[:, k:k + 252, :] for k in range(8)], axis=2)
    win2 = jnp.pad(win2, ((0, 0), (0, 4), (0, 0)))
    h2 = _gelu(jnp.dot(win2.reshape(BT * 256, 512), w2_ref[...],
                       preferred_element_type=f32) + b2_ref[...])
    h2 = h2.reshape(BT, 256, 128)

    h2z = jnp.pad(h2[:, :252, :], ((0, 0), (4, 4), (0, 0)))
    win3 = jnp.concatenate([h2z[:, k:k + 253, :] for k in range(8)], axis=2)
    win3 = jnp.pad(win3, ((0, 0), (0, 3), (0, 0)))
    h3 = _gelu(jnp.dot(win3.reshape(BT * 256, 1024), w3_ref[...],
                       preferred_element_type=f32) + b3_ref[...])
    h3 = h3.reshape(BT, 256, 128)

    h3p = jnp.pad(h3[:, :253, :], ((0, 0), (2, 1), (0, 0)),
                  constant_values=_NEG)
    x1 = jnp.max(h3p.reshape(BT, 64, 4, 128), axis=2)

    xb = xb_ref[...]
    win4 = jnp.concatenate([xb[:, a:a + 31, :] for a in range(5)], axis=2)
    win4 = jnp.pad(win4, ((0, 0), (0, 1), (0, 0)))
    geo = _gelu(jnp.dot(win4.reshape(BT * 32, 500), w4_ref[...],
                        preferred_element_type=f32) + b4_ref[...])
    geo = geo.reshape(BT, 32, 128)

    ge = geo[:, :31, :64]
    go = geo[:, :31, 64:]
    ridx = jax.lax.broadcasted_iota(jnp.int32, (BT, 31, 64), 1)
    m2 = jnp.maximum(ge, jnp.where(ridx < 30, go, _NEG))
    m2p = jnp.pad(m2, ((0, 0), (1, 0), (0, 0)), constant_values=_NEG)
    p4 = jnp.maximum(m2p[:, 0:31, :], m2p[:, 1:32, :])

    p4z = jnp.pad(p4, ((0, 0), (3, 3), (0, 0)))
    win5 = jnp.concatenate([p4z[:, k:k + 31, :] for k in range(7)], axis=2)
    win5 = jnp.pad(win5, ((0, 0), (0, 1), (0, 0)))
    g2 = _gelu(jnp.dot(win5.reshape(BT * 32, 448), w5_ref[...],
                       preferred_element_type=f32) + b5_ref[...])
    g2 = g2.reshape(BT, 32, 128)

    g2z = jnp.pad(g2[:, :31, :], ((0, 0), (3, 3), (0, 0)))
    win6 = jnp.concatenate([g2z[:, k:k + 31, :] for k in range(7)], axis=2)
    win6 = jnp.pad(win6, ((0, 0), (0, 1), (0, 0)))
    g3 = _gelu(jnp.dot(win6.reshape(BT * 32, 896), w6_ref[...],
                       preferred_element_type=f32) + b6_ref[...])
    g3 = g3.reshape(BT, 32, 128)

    g3p = jnp.pad(g3[:, :31, :], ((0, 0), (1, 0), (0, 0)),
                  constant_values=_NEG)
    x2 = jnp.max(g3p.reshape(BT, 16, 2, 128), axis=2)

    out_cl = jnp.concatenate([x1, x2], axis=1)
    o_ref[...] = jnp.transpose(out_cl, (0, 2, 1))


def _fold_bn(gamma, beta, mean, var, eps=1e-5):
    inv = gamma / jnp.sqrt(var + eps)
    return inv, beta - mean * inv


def kernel(x,
           b1c1_w, b1bn1_gamma, b1bn1_beta, b1bn1_mean, b1bn1_var,
           b1c2_w, b1bn2_gamma, b1bn2_beta, b1bn2_mean, b1bn2_var,
           b1c3_w, b1bn3_gamma, b1bn3_beta, b1bn3_mean, b1bn3_var,
           b2c1_w, b2bn1_gamma, b2bn1_beta, b2bn1_mean, b2bn1_var,
           b2c2_w, b2bn2_gamma, b2bn2_beta, b2bn2_mean, b2bn2_var,
           b2c3_w, b2bn3_gamma, b2bn3_beta, b2bn3_mean, b2bn3_var):
    B = x.shape[0]
    x2d = x.reshape(B, 3000)

    wa = jnp.pad(x2d, ((0, 0), (24, 24))).reshape(B, 254, 12)
    xb = jnp.pad(x2d, ((0, 0), (200, 300))).reshape(B, 35, 100)

    s1, t1 = _fold_bn(b1bn1_gamma, b1bn1_beta, b1bn1_mean, b1bn1_var)
    w1 = b1c1_w[:, 0, :] * s1[:, None]
    w1eo = jnp.concatenate([jnp.pad(w1, ((0, 0), (0, 10))).T,
                            jnp.pad(w1, ((0, 0), (6, 4))).T], axis=1)
    b1eo = jnp.concatenate([t1, t1]).reshape(1, 128)

    s2, t2 = _fold_bn(b1bn2_gamma, b1bn2_beta, b1bn2_mean, b1bn2_var)
    w2f = b1c2_w.transpose(2, 1, 0).reshape(512, 128) * s2[None, :]
    b2v = t2.reshape(1, 128)

    s3, t3 = _fold_bn(b1bn3_gamma, b1bn3_beta, b1bn3_mean, b1bn3_var)
    w3f = b1c3_w.transpose(2, 1, 0).reshape(1024, 128) * s3[None, :]
    b3v = t3.reshape(1, 128)

    s4, t4 = _fold_bn(b2bn1_gamma, b2bn1_beta, b2bn1_mean, b2bn1_var)
    w4 = b2c1_w[:, 0, :] * s4[:, None]
    w4eo = jnp.concatenate([jnp.pad(w4, ((0, 0), (0, 100))).T,
                            jnp.pad(w4, ((0, 0), (50, 50))).T], axis=1)
    b4eo = jnp.concatenate([t4, t4]).reshape(1, 128)

    s5, t5 = _fold_bn(b2bn2_gamma, b2bn2_beta, b2bn2_mean, b2bn2_var)
    w5f = b2c2_w.transpose(2, 1, 0).reshape(448, 128) * s5[None, :]
    b5v = t5.reshape(1, 128)

    s6, t6 = _fold_bn(b2bn3_gamma, b2bn3_beta, b2bn3_mean, b2bn3_var)
    w6f = b2c3_w.transpose(2, 1, 0).reshape(896, 128) * s6[None, :]
    b6v = t6.reshape(1, 128)

    bcast = lambda i: (0, 0)
    return pl.pallas_call(
        _fused_kernel,
        out_shape=jax.ShapeDtypeStruct((B, 128, 80), x.dtype),
        grid=(B // _BT,),
        in_specs=[
            pl.BlockSpec((_BT, 254, 12), lambda i: (i, 0, 0)),
            pl.BlockSpec((_BT, 35, 100), lambda i: (i, 0, 0)),
            pl.BlockSpec((60, 128), bcast), pl.BlockSpec((1, 128), bcast),
            pl.BlockSpec((512, 128), bcast), pl.BlockSpec((1, 128), bcast),
            pl.BlockSpec((1024, 128), bcast), pl.BlockSpec((1, 128), bcast),
            pl.BlockSpec((500, 128), bcast), pl.BlockSpec((1, 128), bcast),
            pl.BlockSpec((448, 128), bcast), pl.BlockSpec((1, 128), bcast),
            pl.BlockSpec((896, 128), bcast), pl.BlockSpec((1, 128), bcast),
        ],
        out_specs=pl.BlockSpec((_BT, 128, 80), lambda i: (i, 0, 0)),
        compiler_params=pltpu.CompilerParams(
            dimension_semantics=("parallel",)),
    )(wa, xb, w1eo, b1eo, w2f, b2v, w3f, b3v, w4eo, b4eo, w5f, b5v, w6f, b6v)


# speedup vs baseline: 1.9818x; 1.0427x over previous
import jax
import jax.numpy as jnp
from jax.experimental import pallas as pl
from jax.experimental.pallas import tpu as pltpu

_BT = 8
_NEG = float(jnp.finfo(jnp.float32).min)


def _gelu(x):
    h = 0.5 * x
    return h + h * jax.lax.erf(x * 0.7071067811865476)


def _fused_kernel(xa_ref, xb_ref,
                  w1_ref, b1_ref, w2_ref, b2_ref, w3_ref, b3_ref,
                  w4_ref, b4_ref, w5_ref, b5_ref, w6_ref, b6_ref,
                  o_ref):
    BT = xa_ref.shape[0]
    f32 = jnp.float32

    xa = xa_ref[...]
    win1 = jnp.concatenate([xa[:, a:a + 250, :] for a in range(5)], axis=2)
    win1 = jnp.pad(win1, ((0, 0), (0, 6), (0, 0)))
    heo = jnp.dot(win1.reshape(BT * 256, 60), w1_ref[...],
                  preferred_element_type=f32) + b1_ref[...]
    heo = _gelu(heo).reshape(BT, 256, 128)

    m = jnp.maximum(heo[:, :250, :64], heo[:, :250, 64:])
    mp = jnp.pad(m, ((0, 0), (2, 2), (0, 0)), constant_values=_NEG)
    p1 = mp[:, 0:251, :]
    for a in range(1, 4):
        p1 = jnp.maximum(p1, mp[:, a:a + 251, :])

    p1z = jnp.pad(p1, ((0, 0), (4, 4), (0, 0)))
    win2 = jnp.concatenate([p1z[:, k:k + 252, :] for k in range(8)], axis=2)
    win2 = jnp.pad(win2, ((0, 0), (0, 4), (0, 0)))
    h2 = _gelu(jnp.dot(win2.reshape(BT * 256, 512), w2_ref[...],
                       preferred_element_type=f32) + b2_ref[...])
    h2 = h2.reshape(BT, 256, 128)

    h2z = jnp.pad(h2[:, :252, :], ((0, 0), (4, 4), (0, 0)))
    win3 = jnp.concatenate([h2z[:, k:k + 253, :] for k in range(8)], axis=2)
    win3 = jnp.pad(win3, ((0, 0), (0, 3), (0, 0)))
    h3 = _gelu(jnp.dot(win3.reshape(BT * 256, 1024), w3_ref[...],
                       preferred_element_type=f32) + b3_ref[...])
    h3 = h3.reshape(BT, 256, 128)

    h3p = jnp.pad(h3[:, :253, :], ((0, 0), (2, 1), (0, 0)),
                  constant_values=_NEG)
    x1 = jnp.max(h3p.reshape(BT, 64, 4, 128), axis=2)

    xb = xb_ref[...]
    win4 = jnp.concatenate([xb[:, a:a + 31, :] for a in range(5)], axis=2)
    win4 = jnp.pad(win4, ((0, 0), (0, 1), (0, 0)))
    geo = _gelu(jnp.dot(win4.reshape(BT * 32, 500), w4_ref[...],
                        preferred_element_type=f32) + b4_ref[...])
    geo = geo.reshape(BT, 32, 128)

    ge = geo[:, :31, :64]
    go = geo[:, :31, 64:]
    ridx = jax.lax.broadcasted_iota(jnp.int32, (BT, 31, 64), 1)
    m2 = jnp.maximum(ge, jnp.where(ridx < 30, go, _NEG))
    m2p = jnp.pad(m2, ((0, 0), (1, 0), (0, 0)), constant_values=_NEG)
    p4 = jnp.maximum(m2p[:, 0:31, :], m2p[:, 1:32, :])

    p4z = jnp.pad(p4, ((0, 0), (3, 3), (0, 0)))
    win5 = jnp.concatenate([p4z[:, k:k + 31, :] for k in range(7)], axis=2)
    win5 = jnp.pad(win5, ((0, 0), (0, 1), (0, 0)))
    g2 = _gelu(jnp.dot(win5.reshape(BT * 32, 448), w5_ref[...],
                       preferred_element_type=f32) + b5_ref[...])
    g2 = g2.reshape(BT, 32, 128)

    g2z = jnp.pad(g2[:, :31, :], ((0, 0), (3, 3), (0, 0)))
    win6 = jnp.concatenate([g2z[:, k:k + 31, :] for k in range(7)], axis=2)
    win6 = jnp.pad(win6, ((0, 0), (0, 1), (0, 0)))
    g3 = _gelu(jnp.dot(win6.reshape(BT * 32, 896), w6_ref[...],
                       preferred_element_type=f32) + b6_ref[...])
    g3 = g3.reshape(BT, 32, 128)

    g3p = jnp.pad(g3[:, :31, :], ((0, 0), (1, 0), (0, 0)),
                  constant_values=_NEG)
    x2 = jnp.max(g3p.reshape(BT, 16, 2, 128), axis=2)

    out_cl = jnp.concatenate([x1, x2], axis=1)
    o_ref[...] = jnp.transpose(out_cl, (0, 2, 1))


def _fold_bn(gamma, beta, mean, var, eps=1e-5):
    inv = gamma / jnp.sqrt(var + eps)
    return inv, beta - mean * inv


def kernel(x,
           b1c1_w, b1bn1_gamma, b1bn1_beta, b1bn1_mean, b1bn1_var,
           b1c2_w, b1bn2_gamma, b1bn2_beta, b1bn2_mean, b1bn2_var,
           b1c3_w, b1bn3_gamma, b1bn3_beta, b1bn3_mean, b1bn3_var,
           b2c1_w, b2bn1_gamma, b2bn1_beta, b2bn1_mean, b2bn1_var,
           b2c2_w, b2bn2_gamma, b2bn2_beta, b2bn2_mean, b2bn2_var,
           b2c3_w, b2bn3_gamma, b2bn3_beta, b2bn3_mean, b2bn3_var):
    B = x.shape[0]
    x2d = x.reshape(B, 3000)

    wa = jnp.pad(x2d, ((0, 0), (24, 24))).reshape(B, 254, 12)
    xb = jnp.pad(x2d, ((0, 0), (200, 300))).reshape(B, 35, 100)

    s1, t1 = _fold_bn(b1bn1_gamma, b1bn1_beta, b1bn1_mean, b1bn1_var)
    w1 = b1c1_w[:, 0, :] * s1[:, None]
    w1eo = jnp.concatenate([jnp.pad(w1, ((0, 0), (0, 10))).T,
                            jnp.pad(w1, ((0, 0), (6, 4))).T], axis=1)
    b1eo = jnp.concatenate([t1, t1]).reshape(1, 128)

    s2, t2 = _fold_bn(b1bn2_gamma, b1bn2_beta, b1bn2_mean, b1bn2_var)
    w2f = b1c2_w.transpose(2, 1, 0).reshape(512, 128) * s2[None, :]
    b2v = t2.reshape(1, 128)

    s3, t3 = _fold_bn(b1bn3_gamma, b1bn3_beta, b1bn3_mean, b1bn3_var)
    w3f = b1c3_w.transpose(2, 1, 0).reshape(1024, 128) * s3[None, :]
    b3v = t3.reshape(1, 128)

    s4, t4 = _fold_bn(b2bn1_gamma, b2bn1_beta, b2bn1_mean, b2bn1_var)
    w4 = b2c1_w[:, 0, :] * s4[:, None]
    w4eo = jnp.concatenate([jnp.pad(w4, ((0, 0), (0, 100))).T,
                            jnp.pad(w4, ((0, 0), (50, 50))).T], axis=1)
    b4eo = jnp.concatenate([t4, t4]).reshape(1, 128)

    s5, t5 = _fold_bn(b2bn2_gamma, b2bn2_beta, b2bn2_mean, b2bn2_var)
    w5f = b2c2_w.transpose(2, 1, 0).reshape(448, 128) * s5[None, :]
    b5v = t5.reshape(1, 128)

    s6, t6 = _fold_bn(b2bn3_gamma, b2bn3_beta, b2bn3_mean, b2bn3_var)
    w6f = b2c3_w.transpose(2, 1, 0).reshape(896, 128) * s6[None, :]
    b6v = t6.reshape(1, 128)

    bcast = lambda i: (0, 0)
    return pl.pallas_call(
        _fused_kernel,
        out_shape=jax.ShapeDtypeStruct((B, 128, 80), x.dtype),
        grid=(B // _BT,),
        in_specs=[
            pl.BlockSpec((_BT, 254, 12), lambda i: (i, 0, 0)),
            pl.BlockSpec((_BT, 35, 100), lambda i: (i, 0, 0)),
            pl.BlockSpec((60, 128), bcast), pl.BlockSpec((1, 128), bcast),
            pl.BlockSpec((512, 128), bcast), pl.BlockSpec((1, 128), bcast),
            pl.BlockSpec((1024, 128), bcast), pl.BlockSpec((1, 128), bcast),
            pl.BlockSpec((500, 128), bcast), pl.BlockSpec((1, 128), bcast),
            pl.BlockSpec((448, 128), bcast), pl.BlockSpec((1, 128), bcast),
            pl.BlockSpec((896, 128), bcast), pl.BlockSpec((1, 128), bcast),
        ],
        out_specs=pl.BlockSpec((_BT, 128, 80), lambda i: (i, 0, 0)),
        compiler_params=pltpu.CompilerParams(
            dimension_semantics=("parallel",)),
    )(wa, xb, w1eo, b1eo, w2f, b2v, w3f, b3v, w4eo, b4eo, w5f, b5v, w6f, b6v)


# speedup vs baseline: 2.0276x; 1.0231x over previous
import jax
import jax.numpy as jnp
from jax.experimental import pallas as pl
from jax.experimental.pallas import tpu as pltpu

_BT = 8
_NEG = float(jnp.finfo(jnp.float32).min)

_W1, _W2, _W3, _W4, _W5, _W6, _BIAS = 0, 60, 572, 1596, 2096, 2544, 3440


def _gelu(x):
    h = 0.5 * x
    return h + h * jax.lax.erf(x * 0.7071067811865476)


def _fused_kernel(xa_ref, xb_ref, w_ref, o_ref):
    BT = xa_ref.shape[0]
    f32 = jnp.float32

    win1 = jnp.concatenate(
        [xa_ref[:, a:a + 250, :] for a in range(5)], axis=2)
    win1 = jnp.pad(win1, ((0, 0), (0, 6), (0, 0)))
    heo = jnp.dot(win1.reshape(BT * 256, 60), w_ref[_W1:_W1 + 60, :],
                  preferred_element_type=f32) + w_ref[_BIAS:_BIAS + 1, :]
    heo = _gelu(heo).reshape(BT, 256, 128)

    m = jnp.maximum(heo[:, :250, :64], heo[:, :250, 64:])
    mp = jnp.pad(m, ((0, 0), (2, 2), (0, 0)), constant_values=_NEG)
    p1 = mp[:, 0:251, :]
    for a in range(1, 4):
        p1 = jnp.maximum(p1, mp[:, a:a + 251, :])

    p1z = jnp.pad(p1, ((0, 0), (4, 4), (0, 0)))
    win2 = jnp.concatenate([p1z[:, k:k + 252, :] for k in range(8)], axis=2)
    win2 = jnp.pad(win2, ((0, 0), (0, 4), (0, 0)))
    h2 = _gelu(jnp.dot(win2.reshape(BT * 256, 512), w_ref[_W2:_W2 + 512, :],
                       preferred_element_type=f32) + w_ref[_BIAS + 1:_BIAS + 2, :])
    h2 = h2.reshape(BT, 256, 128)

    h2z = jnp.pad(h2[:, :252, :], ((0, 0), (4, 4), (0, 0)))
    win3 = jnp.concatenate([h2z[:, k:k + 253, :] for k in range(8)], axis=2)
    win3 = jnp.pad(win3, ((0, 0), (0, 3), (0, 0)))
    h3 = _gelu(jnp.dot(win3.reshape(BT * 256, 1024), w_ref[_W3:_W3 + 1024, :],
                       preferred_element_type=f32) + w_ref[_BIAS + 2:_BIAS + 3, :])
    h3 = h3.reshape(BT, 256, 128)

    h3p = jnp.pad(h3[:, :253, :], ((0, 0), (2, 1), (0, 0)),
                  constant_values=_NEG)
    x1 = jnp.max(h3p.reshape(BT, 64, 4, 128), axis=2)

    win4 = jnp.concatenate(
        [xb_ref[:, a:a + 31, :] for a in range(5)], axis=2)
    win4 = jnp.pad(win4, ((0, 0), (0, 1), (0, 0)))
    geo = _gelu(jnp.dot(win4.reshape(BT * 32, 500), w_ref[_W4:_W4 + 500, :],
                        preferred_element_type=f32) + w_ref[_BIAS + 3:_BIAS + 4, :])
    geo = geo.reshape(BT, 32, 128)

    ge = geo[:, :31, :64]
    go = geo[:, :31, 64:]
    ridx = jax.lax.broadcasted_iota(jnp.int32, (BT, 31, 64), 1)
    m2 = jnp.maximum(ge, jnp.where(ridx < 30, go, _NEG))
    m2p = jnp.pad(m2, ((0, 0), (1, 0), (0, 0)), constant_values=_NEG)
    p4 = jnp.maximum(m2p[:, 0:31, :], m2p[:, 1:32, :])

    p4z = jnp.pad(p4, ((0, 0), (3, 3), (0, 0)))
    win5 = jnp.concatenate([p4z[:, k:k + 31, :] for k in range(7)], axis=2)
    win5 = jnp.pad(win5, ((0, 0), (0, 1), (0, 0)))
    g2 = _gelu(jnp.dot(win5.reshape(BT * 32, 448), w_ref[_W5:_W5 + 448, :],
                       preferred_element_type=f32) + w_ref[_BIAS + 4:_BIAS + 5, :])
    g2 = g2.reshape(BT, 32, 128)

    g2z = jnp.pad(g2[:, :31, :], ((0, 0), (3, 3), (0, 0)))
    win6 = jnp.concatenate([g2z[:, k:k + 31, :] for k in range(7)], axis=2)
    win6 = jnp.pad(win6, ((0, 0), (0, 1), (0, 0)))
    g3 = _gelu(jnp.dot(win6.reshape(BT * 32, 896), w_ref[_W6:_W6 + 896, :],
                       preferred_element_type=f32) + w_ref[_BIAS + 5:_BIAS + 6, :])
    g3 = g3.reshape(BT, 32, 128)

    g3p = jnp.pad(g3[:, :31, :], ((0, 0), (1, 0), (0, 0)),
                  constant_values=_NEG)
    x2 = jnp.max(g3p.reshape(BT, 16, 2, 128), axis=2)

    out_cl = jnp.concatenate([x1, x2], axis=1)
    o_ref[...] = jnp.transpose(out_cl, (0, 2, 1))


def _fold_bn(gamma, beta, mean, var, eps=1e-5):
    inv = gamma / jnp.sqrt(var + eps)
    return inv, beta - mean * inv


def kernel(x,
           b1c1_w, b1bn1_gamma, b1bn1_beta, b1bn1_mean, b1bn1_var,
           b1c2_w, b1bn2_gamma, b1bn2_beta, b1bn2_mean, b1bn2_var,
           b1c3_w, b1bn3_gamma, b1bn3_beta, b1bn3_mean, b1bn3_var,
           b2c1_w, b2bn1_gamma, b2bn1_beta, b2bn1_mean, b2bn1_var,
           b2c2_w, b2bn2_gamma, b2bn2_beta, b2bn2_mean, b2bn2_var,
           b2c3_w, b2bn3_gamma, b2bn3_beta, b2bn3_mean, b2bn3_var):
    B = x.shape[0]
    x2d = x.reshape(B, 3000)

    wa = jnp.pad(x2d, ((0, 0), (24, 24))).reshape(B, 254, 12)
    xb = jnp.pad(x2d, ((0, 0), (200, 300))).reshape(B, 35, 100)

    s1, t1 = _fold_bn(b1bn1_gamma, b1bn1_beta, b1bn1_mean, b1bn1_var)
    w1 = b1c1_w[:, 0, :] * s1[:, None]
    w1eo = jnp.concatenate([jnp.pad(w1, ((0, 0), (0, 10))).T,
                            jnp.pad(w1, ((0, 0), (6, 4))).T], axis=1)

    s2, t2 = _fold_bn(b1bn2_gamma, b1bn2_beta, b1bn2_mean, b1bn2_var)
    w2f = b1c2_w.transpose(2, 1, 0).reshape(512, 128) * s2[None, :]

    s3, t3 = _fold_bn(b1bn3_gamma, b1bn3_beta, b1bn3_mean, b1bn3_var)
    w3f = b1c3_w.transpose(2, 1, 0).reshape(1024, 128) * s3[None, :]

    s4, t4 = _fold_bn(b2bn1_gamma, b2bn1_beta, b2bn1_mean, b2bn1_var)
    w4 = b2c1_w[:, 0, :] * s4[:, None]
    w4eo = jnp.concatenate([jnp.pad(w4, ((0, 0), (0, 100))).T,
                            jnp.pad(w4, ((0, 0), (50, 50))).T], axis=1)

    s5, t5 = _fold_bn(b2bn2_gamma, b2bn2_beta, b2bn2_mean, b2bn2_var)
    w5f = b2c2_w.transpose(2, 1, 0).reshape(448, 128) * s5[None, :]

    s6, t6 = _fold_bn(b2bn3_gamma, b2bn3_beta, b2bn3_mean, b2bn3_var)
    w6f = b2c3_w.transpose(2, 1, 0).reshape(896, 128) * s6[None, :]

    biases = jnp.stack([jnp.concatenate([t1, t1]), t2, t3,
                        jnp.concatenate([t4, t4]), t5, t6])
    wpk = jnp.concatenate(
        [w1eo, w2f, w3f, w4eo, w5f, w6f, biases,
         jnp.zeros((2, 128), x.dtype)], axis=0)

    return pl.pallas_call(
        _fused_kernel,
        out_shape=jax.ShapeDtypeStruct((B, 128, 80), x.dtype),
        grid=(B // _BT,),
        in_specs=[
            pl.BlockSpec((_BT, 254, 12), lambda i: (i, 0, 0)),
            pl.BlockSpec((_BT, 35, 100), lambda i: (i, 0, 0)),
            pl.BlockSpec((3448, 128), lambda i: (0, 0)),
        ],
        out_specs=pl.BlockSpec((_BT, 128, 80), lambda i: (i, 0, 0)),
        compiler_params=pltpu.CompilerParams(
            dimension_semantics=("parallel",)),
    )(wa, xb, wpk)
